# baseline (speedup 1.0000x reference)
import functools

import numpy as np
import jax
import jax.numpy as jnp
from jax import lax
from jax.experimental import pallas as pl
from jax.experimental.pallas import tpu as pltpu
from jax.experimental.pallas import tpu_sc as plsc

T = 2048
D = 768
F = 1280
E = 8
K = 2
A = T * K
MG = 512
RBUF = A + E * MG
NT = RBUF // MG
NSH = T // MG
NS = NT + NSH
NW = 32
CHUNK = A // NW
MF = 512
NFC = 5
FC = F // NFC

_P, _Q = 32, 128
_C = _P * E
_cc = np.arange(_C)
_REP = (np.arange(_P)[:, None] == (_cc // E)[None, :]).astype(np.float32)
_EIDX = (_cc % E).astype(np.float32)[None, :]
_R256 = (((_cc % E)[:, None] == (_cc % E)[None, :])
         & ((_cc // E)[:, None] < (_cc // E)[None, :])).astype(np.float32)
_RSUM = ((_cc % E)[:, None] == np.arange(E)[None, :]).astype(np.float32)
_L128 = (np.arange(_Q)[None, :] < np.arange(_Q)[:, None]).astype(np.float32)
_L8 = (np.arange(E)[:, None] < np.arange(E)[None, :]).astype(np.float32)
_TILE8 = (np.arange(E)[:, None] == (_cc % E)[None, :]).astype(np.float32)
_REPT = ((_cc // E)[:, None] == np.arange(_P)[None, :]).astype(np.float32)

_GSTART = (np.arange(NT) * MG).astype(np.float32)[:, None]


RT = 512
NRT = T // RT


def _route_index_body(x_ref, rw_ref, rep_ref, eidx_ref, r_ref, rsum_ref,
                      l128_ref, l8_ref, tile8_ref, rept_ref, gs_ref,
                      pos_ref, ex_ref, val_ref, w1_ref, w2_ref,
                      e1_s, e2_s):
    i = pl.program_id(0)

    @pl.when(i < NRT)
    def _router():
        x = x_ref[...]
        logits = jnp.dot(x, rw_ref[...], preferred_element_type=jnp.float32)
        aff = jax.nn.sigmoid(logits)
        ii = jax.lax.broadcasted_iota(jnp.int32, aff.shape, 1)
        m1 = jnp.max(aff, axis=1, keepdims=True)
        e1 = jnp.min(jnp.where(aff == m1, ii, E), axis=1, keepdims=True)
        aff2 = jnp.where(ii == e1, -1.0, aff)
        m2 = jnp.max(aff2, axis=1, keepdims=True)
        e2 = jnp.min(jnp.where(aff2 == m2, ii, E), axis=1, keepdims=True)
        denom = m1 + m2
        w1_ref[pl.ds(i * RT, RT), :] = m1 / denom
        w2_ref[pl.ds(i * RT, RT), :] = m2 / denom
        e1_s[pl.ds(i * RT, RT), :] = e1
        e2_s[pl.ds(i * RT, RT), :] = e2

    @pl.when(i == NRT)
    def _index():
        _index_phase(e1_s, e2_s, rep_ref, eidx_ref, r_ref, rsum_ref,
                     l128_ref, l8_ref, tile8_ref, rept_ref, gs_ref,
                     pos_ref, ex_ref, val_ref)


def _index_phase(e1_s, e2_s, rep_ref, eidx_ref, r_ref, rsum_ref,
                 l128_ref, l8_ref, tile8_ref, rept_ref, gs_ref,
                 pos_ref, ex_ref, val_ref):
    e1 = e1_s[...]
    e2 = e2_s[...]
    hi = jax.lax.Precision.HIGHEST
    dot = functools.partial(jnp.dot, precision=hi,
                            preferred_element_type=jnp.float32)
    ef = jnp.concatenate(
        [jnp.reshape(e1, (_Q // 2, _P)), jnp.reshape(e2, (_Q // 2, _P))],
        axis=0).astype(jnp.float32)
    eexp = dot(ef, rep_ref[...])
    z = (eexp == eidx_ref[...]).astype(jnp.float32)
    intra = dot(z, r_ref[...])
    rowcnt = dot(z, rsum_ref[...])
    rowoff = dot(l128_ref[...], rowcnt)
    counts = jnp.sum(rowcnt, axis=0, keepdims=True)
    pc = jnp.floor((counts + (MG - 1)) * (1.0 / MG)) * MG
    pad_off = dot(pc, l8_ref[...])
    off_exp = dot(rowoff + pad_off, tile8_ref[...])
    pos_sel = dot(z * (intra + off_exp), rept_ref[...])
    pos_ref[...] = pos_sel.astype(jnp.int32)

    pad_end = pad_off + pc
    gs = gs_ref[...]
    ex = jnp.sum((gs >= pad_end).astype(jnp.float32), axis=1, keepdims=True)
    ex_ref[...] = jnp.clip(ex, 0.0, float(E - 1)).astype(jnp.int32)
    val_ref[...] = (gs < pad_end[:, E - 1:E]).astype(jnp.int32)


@functools.lru_cache(maxsize=1)
def _sc_kernels():
    mesh = plsc.VectorSubcoreMesh(core_axis_name="c", subcore_axis_name="s")

    H = CHUNK // 2

    @functools.partial(
        pl.kernel,
        out_type=jax.ShapeDtypeStruct((RBUF, D), jnp.float32),
        mesh=mesh,
        scratch_types=[
            pltpu.VMEM((2, H), jnp.int32),
            pltpu.VMEM((2, H, D), jnp.float32),
            pltpu.SemaphoreType.DMA,
            pltpu.SemaphoreType.DMA,
            pltpu.SemaphoreType.DMA,
        ],
    )
    def _dispatch(pos_hbm, x_hbm, xs_hbm, pos_v, rows_v, psem, gsem, ssem):
        wid = lax.axis_index("s") * 2 + lax.axis_index("c")
        base = wid * CHUNK
        tbase = lax.rem(base, T)
        cp0 = pltpu.async_copy(pos_hbm.at[pl.ds(base, H)], pos_v.at[0], psem)
        cp1 = pltpu.async_copy(pos_hbm.at[pl.ds(base + H, H)], pos_v.at[1],
                               psem)
        g0 = pltpu.async_copy(x_hbm.at[pl.ds(tbase, H)], rows_v.at[0], gsem)
        g1 = pltpu.async_copy(x_hbm.at[pl.ds(tbase + H, H)], rows_v.at[1],
                              gsem)
        cp0.wait()
        g0.wait()
        s0 = pltpu.async_copy(rows_v.at[0], xs_hbm.at[pos_v.at[0]], ssem)
        cp1.wait()
        g1.wait()
        s1 = pltpu.async_copy(rows_v.at[1], xs_hbm.at[pos_v.at[1]], ssem)
        s0.wait()
        s1.wait()

    @functools.partial(
        pl.kernel,
        out_type=jax.ShapeDtypeStruct((A, D), jnp.float32),
        mesh=mesh,
        scratch_types=[
            pltpu.VMEM((2, H), jnp.int32),
            pltpu.VMEM((2, H, D), jnp.float32),
            pltpu.SemaphoreType.DMA,
            pltpu.SemaphoreType.DMA,
            pltpu.SemaphoreType.DMA,
        ],
    )
    def _collect(pos_hbm, y_hbm, yall_hbm, pos_v, rows_v, psem, gsem, ssem):
        wid = lax.axis_index("s") * 2 + lax.axis_index("c")
        base = wid * CHUNK
        cp0 = pltpu.async_copy(pos_hbm.at[pl.ds(base, H)], pos_v.at[0], psem)
        cp1 = pltpu.async_copy(pos_hbm.at[pl.ds(base + H, H)], pos_v.at[1],
                               psem)
        cp0.wait()
        g0 = pltpu.async_copy(y_hbm.at[pos_v.at[0]], rows_v.at[0], gsem)
        cp1.wait()
        g1 = pltpu.async_copy(y_hbm.at[pos_v.at[1]], rows_v.at[1], gsem)
        g0.wait()
        s0 = pltpu.async_copy(rows_v.at[0],
                              yall_hbm.at[pl.ds(base, H)], ssem)
        g1.wait()
        s1 = pltpu.async_copy(rows_v.at[1],
                              yall_hbm.at[pl.ds(base + H, H)], ssem)
        s0.wait()
        s1.wait()

    return _dispatch, _collect


def _grouped_body(ex_ref, val_ref, xs_ref, wg_ref, wu_ref, wd_ref, y_ref):
    s = pl.program_id(0)

    @pl.when(val_ref[s, 0] == 1)
    def _expert():
        x = xs_ref[...]
        g = jax.nn.silu(jnp.dot(x, wg_ref[0],
                                preferred_element_type=jnp.float32))
        u = jnp.dot(x, wu_ref[0], preferred_element_type=jnp.float32)
        y_ref[...] = jnp.dot(g * u, wd_ref[0],
                             preferred_element_type=jnp.float32)


def _shared_body(x_ref, wsg_ref, wsu_ref, wsd_ref, out_ref):
    x = x_ref[...]
    g = jax.nn.silu(jnp.dot(x, wsg_ref[...],
                            preferred_element_type=jnp.float32))
    u = jnp.dot(x, wsu_ref[...], preferred_element_type=jnp.float32)
    out_ref[...] = jnp.dot(g * u, wsd_ref[...],
                           preferred_element_type=jnp.float32)


def _final_body(sh_ref, y0_ref, y1_ref, w1_ref, w2_ref, out_ref):
    out_ref[...] = (sh_ref[...] + w1_ref[...] * y0_ref[...]
                    + w2_ref[...] * y1_ref[...])


@jax.jit
def kernel(x, router_w, W_gate, W_up, W_down, Ws_gate, Ws_up, Ws_down):
    full = lambda s: pl.BlockSpec(s, lambda i: tuple(0 for _ in s))
    pos_qp, ex_tile, val_tile, w1, w2 = pl.pallas_call(
        _route_index_body,
        grid=(NRT + 1,),
        in_specs=[pl.BlockSpec((RT, D), lambda i: (jnp.minimum(i, NRT - 1), 0)),
                  full((D, E)), full((_P, _C)), full((1, _C)),
                  full((_C, _C)), full((_C, E)), full((_Q, _Q)),
                  full((E, E)), full((E, _C)), full((_C, _P)),
                  full((NT, 1))],
        out_specs=[full((_Q, _P)), full((NT, 1)), full((NT, 1)),
                   full((T, 1)), full((T, 1))],
        out_shape=[
            jax.ShapeDtypeStruct((_Q, _P), jnp.int32),
            jax.ShapeDtypeStruct((NT, 1), jnp.int32),
            jax.ShapeDtypeStruct((NT, 1), jnp.int32),
            jax.ShapeDtypeStruct((T, 1), jnp.float32),
            jax.ShapeDtypeStruct((T, 1), jnp.float32),
        ],
        scratch_shapes=[
            pltpu.VMEM((T, 1), jnp.int32),
            pltpu.VMEM((T, 1), jnp.int32),
        ],
        compiler_params=pltpu.CompilerParams(
            dimension_semantics=("arbitrary",),
        ),
    )(x, router_w, _REP, _EIDX, _R256, _RSUM, _L128, _L8, _TILE8, _REPT,
      _GSTART)
    pos_flat = pos_qp.reshape(A)

    sh = pl.pallas_call(
        _shared_body,
        grid=(T // MF,),
        in_specs=[
            pl.BlockSpec((MF, D), lambda i: (i, 0)),
            pl.BlockSpec((D, F), lambda i: (0, 0)),
            pl.BlockSpec((D, F), lambda i: (0, 0)),
            pl.BlockSpec((F, D), lambda i: (0, 0)),
        ],
        out_specs=pl.BlockSpec((MF, D), lambda i: (i, 0)),
        out_shape=jax.ShapeDtypeStruct((T, D), jnp.float32),
        compiler_params=pltpu.CompilerParams(
            dimension_semantics=("parallel",),
        ),
    )(x, Ws_gate, Ws_up, Ws_down)

    _dispatch, _collect = _sc_kernels()
    xs = _dispatch(pos_flat, x)

    y = pl.pallas_call(
        _grouped_body,
        grid_spec=pltpu.PrefetchScalarGridSpec(
            num_scalar_prefetch=2,
            grid=(NT,),
            in_specs=[
                pl.BlockSpec((MG, D), lambda i, ex, vl: (i, 0)),
                pl.BlockSpec((1, D, F), lambda i, ex, vl: (ex[i, 0], 0, 0)),
                pl.BlockSpec((1, D, F), lambda i, ex, vl: (ex[i, 0], 0, 0)),
                pl.BlockSpec((1, F, D), lambda i, ex, vl: (ex[i, 0], 0, 0)),
            ],
            out_specs=pl.BlockSpec((MG, D), lambda i, ex, vl: (i, 0)),
        ),
        out_shape=jax.ShapeDtypeStruct((RBUF, D), jnp.float32),
        compiler_params=pltpu.CompilerParams(
            dimension_semantics=("arbitrary",),
        ),
    )(ex_tile, val_tile, xs, W_gate, W_up, W_down)

    yall = _collect(pos_flat, y)

    nf = T // MF
    out = pl.pallas_call(
        _final_body,
        grid=(nf,),
        in_specs=[
            pl.BlockSpec((MF, D), lambda i: (i, 0)),
            pl.BlockSpec((MF, D), lambda i: (i, 0)),
            pl.BlockSpec((MF, D), lambda i: (i + nf, 0)),
            pl.BlockSpec((MF, 1), lambda i: (i, 0)),
            pl.BlockSpec((MF, 1), lambda i: (i, 0)),
        ],
        out_specs=pl.BlockSpec((MF, D), lambda i: (i, 0)),
        out_shape=jax.ShapeDtypeStruct((T, D), jnp.float32),
        compiler_params=pltpu.CompilerParams(
            dimension_semantics=("parallel",),
        ),
    )(sh, yall, yall, w1, w2)
    return out


# speedup vs baseline: 1.0048x; 1.0048x over previous
import functools

import numpy as np
import jax
import jax.numpy as jnp
from jax import lax
from jax.experimental import pallas as pl
from jax.experimental.pallas import tpu as pltpu
from jax.experimental.pallas import tpu_sc as plsc

T = 2048
D = 768
F = 1280
E = 8
K = 2
A = T * K
MG = 512
RBUF = A + E * MG
NT = RBUF // MG
NSH = T // MG
NS = NT + NSH
NW = 32
CHUNK = A // NW
MF = 512
NFC = 5
FC = F // NFC

_P, _Q = 32, 128
_C = _P * E
_cc = np.arange(_C)
_REP = (np.arange(_P)[:, None] == (_cc // E)[None, :]).astype(np.float32)
_EIDX = (_cc % E).astype(np.float32)[None, :]
_R256 = (((_cc % E)[:, None] == (_cc % E)[None, :])
         & ((_cc // E)[:, None] < (_cc // E)[None, :])).astype(np.float32)
_RSUM = ((_cc % E)[:, None] == np.arange(E)[None, :]).astype(np.float32)
_L128 = (np.arange(_Q)[None, :] < np.arange(_Q)[:, None]).astype(np.float32)
_L8 = (np.arange(E)[:, None] < np.arange(E)[None, :]).astype(np.float32)
_TILE8 = (np.arange(E)[:, None] == (_cc % E)[None, :]).astype(np.float32)
_REPT = ((_cc // E)[:, None] == np.arange(_P)[None, :]).astype(np.float32)

_GSTART = (np.arange(NT) * MG).astype(np.float32)[:, None]


def _route_index_body(x_ref, rw_ref, rep_ref, eidx_ref, r_ref, rsum_ref,
                      l128_ref, l8_ref, tile8_ref, rept_ref, gs_ref,
                      pos_ref, ex_ref, val_ref, w1_ref, w2_ref):
    x = x_ref[...]
    logits = jnp.dot(x, rw_ref[...], preferred_element_type=jnp.float32)
    aff = jax.nn.sigmoid(logits)
    ii = jax.lax.broadcasted_iota(jnp.int32, aff.shape, 1)
    m1 = jnp.max(aff, axis=1, keepdims=True)
    e1 = jnp.min(jnp.where(aff == m1, ii, E), axis=1, keepdims=True)
    aff2 = jnp.where(ii == e1, -1.0, aff)
    m2 = jnp.max(aff2, axis=1, keepdims=True)
    e2 = jnp.min(jnp.where(aff2 == m2, ii, E), axis=1, keepdims=True)
    denom = m1 + m2
    w1_ref[...] = m1 / denom
    w2_ref[...] = m2 / denom

    hi = jax.lax.Precision.HIGHEST
    dot = functools.partial(jnp.dot, precision=hi,
                            preferred_element_type=jnp.float32)
    ef = jnp.concatenate(
        [jnp.reshape(e1, (_Q // 2, _P)), jnp.reshape(e2, (_Q // 2, _P))],
        axis=0).astype(jnp.float32)
    eexp = dot(ef, rep_ref[...])
    z = (eexp == eidx_ref[...]).astype(jnp.float32)
    intra = dot(z, r_ref[...])
    rowcnt = dot(z, rsum_ref[...])
    rowoff = dot(l128_ref[...], rowcnt)
    counts = jnp.sum(rowcnt, axis=0, keepdims=True)
    pc = jnp.floor((counts + (MG - 1)) * (1.0 / MG)) * MG
    pad_off = dot(pc, l8_ref[...])
    off_exp = dot(rowoff + pad_off, tile8_ref[...])
    pos_sel = dot(z * (intra + off_exp), rept_ref[...])
    pos_ref[...] = pos_sel.astype(jnp.int32)

    pad_end = pad_off + pc
    gs = gs_ref[...]
    ex = jnp.sum((gs >= pad_end).astype(jnp.float32), axis=1, keepdims=True)
    ex_ref[...] = jnp.clip(ex, 0.0, float(E - 1)).astype(jnp.int32)
    val_ref[...] = (gs < pad_end[:, E - 1:E]).astype(jnp.int32)


@functools.lru_cache(maxsize=1)
def _sc_kernels():
    mesh = plsc.VectorSubcoreMesh(core_axis_name="c", subcore_axis_name="s")

    H = CHUNK // 2

    @functools.partial(
        pl.kernel,
        out_type=jax.ShapeDtypeStruct((RBUF, D), jnp.float32),
        mesh=mesh,
        scratch_types=[
            pltpu.VMEM((2, H), jnp.int32),
            pltpu.VMEM((2, H, D), jnp.float32),
            pltpu.SemaphoreType.DMA,
            pltpu.SemaphoreType.DMA,
            pltpu.SemaphoreType.DMA,
        ],
    )
    def _dispatch(pos_hbm, x_hbm, xs_hbm, pos_v, rows_v, psem, gsem, ssem):
        wid = lax.axis_index("s") * 2 + lax.axis_index("c")
        base = wid * CHUNK
        tbase = lax.rem(base, T)
        cp0 = pltpu.async_copy(pos_hbm.at[pl.ds(base, H)], pos_v.at[0], psem)
        cp1 = pltpu.async_copy(pos_hbm.at[pl.ds(base + H, H)], pos_v.at[1],
                               psem)
        g0 = pltpu.async_copy(x_hbm.at[pl.ds(tbase, H)], rows_v.at[0], gsem)
        g1 = pltpu.async_copy(x_hbm.at[pl.ds(tbase + H, H)], rows_v.at[1],
                              gsem)
        cp0.wait()
        g0.wait()
        s0 = pltpu.async_copy(rows_v.at[0], xs_hbm.at[pos_v.at[0]], ssem)
        cp1.wait()
        g1.wait()
        s1 = pltpu.async_copy(rows_v.at[1], xs_hbm.at[pos_v.at[1]], ssem)
        s0.wait()
        s1.wait()

    @functools.partial(
        pl.kernel,
        out_type=jax.ShapeDtypeStruct((A, D), jnp.float32),
        mesh=mesh,
        scratch_types=[
            pltpu.VMEM((2, H), jnp.int32),
            pltpu.VMEM((2, H, D), jnp.float32),
            pltpu.SemaphoreType.DMA,
            pltpu.SemaphoreType.DMA,
            pltpu.SemaphoreType.DMA,
        ],
    )
    def _collect(pos_hbm, y_hbm, yall_hbm, pos_v, rows_v, psem, gsem, ssem):
        wid = lax.axis_index("s") * 2 + lax.axis_index("c")
        base = wid * CHUNK
        cp0 = pltpu.async_copy(pos_hbm.at[pl.ds(base, H)], pos_v.at[0], psem)
        cp1 = pltpu.async_copy(pos_hbm.at[pl.ds(base + H, H)], pos_v.at[1],
                               psem)
        cp0.wait()
        g0 = pltpu.async_copy(y_hbm.at[pos_v.at[0]], rows_v.at[0], gsem)
        cp1.wait()
        g1 = pltpu.async_copy(y_hbm.at[pos_v.at[1]], rows_v.at[1], gsem)
        g0.wait()
        s0 = pltpu.async_copy(rows_v.at[0],
                              yall_hbm.at[pl.ds(base, H)], ssem)
        g1.wait()
        s1 = pltpu.async_copy(rows_v.at[1],
                              yall_hbm.at[pl.ds(base + H, H)], ssem)
        s0.wait()
        s1.wait()

    return _dispatch, _collect


def _grouped_body(ex_ref, val_ref, xs_ref, wg_ref, wu_ref, wd_ref, y_ref):
    s = pl.program_id(0)

    @pl.when(val_ref[s, 0] == 1)
    def _expert():
        x = xs_ref[...]
        g = jax.nn.silu(jnp.dot(x, wg_ref[0],
                                preferred_element_type=jnp.float32))
        u = jnp.dot(x, wu_ref[0], preferred_element_type=jnp.float32)
        y_ref[...] = jnp.dot(g * u, wd_ref[0],
                             preferred_element_type=jnp.float32)


def _shared_body(x_ref, wsg_ref, wsu_ref, wsd_ref, out_ref):
    x = x_ref[...]
    g = jax.nn.silu(jnp.dot(x, wsg_ref[...],
                            preferred_element_type=jnp.float32))
    u = jnp.dot(x, wsu_ref[...], preferred_element_type=jnp.float32)
    out_ref[...] = jnp.dot(g * u, wsd_ref[...],
                           preferred_element_type=jnp.float32)


def _final_body(sh_ref, y0_ref, y1_ref, w1_ref, w2_ref, out_ref):
    out_ref[...] = (sh_ref[...] + w1_ref[...] * y0_ref[...]
                    + w2_ref[...] * y1_ref[...])


@jax.jit
def kernel(x, router_w, W_gate, W_up, W_down, Ws_gate, Ws_up, Ws_down):
    full = lambda s: pl.BlockSpec(s, lambda i: tuple(0 for _ in s))
    pos_qp, ex_tile, val_tile, w1, w2 = pl.pallas_call(
        _route_index_body,
        grid=(1,),
        in_specs=[full((T, D)), full((D, E)), full((_P, _C)), full((1, _C)),
                  full((_C, _C)), full((_C, E)), full((_Q, _Q)),
                  full((E, E)), full((E, _C)), full((_C, _P)),
                  full((NT, 1))],
        out_specs=[full((_Q, _P)), full((NT, 1)), full((NT, 1)),
                   full((T, 1)), full((T, 1))],
        out_shape=[
            jax.ShapeDtypeStruct((_Q, _P), jnp.int32),
            jax.ShapeDtypeStruct((NT, 1), jnp.int32),
            jax.ShapeDtypeStruct((NT, 1), jnp.int32),
            jax.ShapeDtypeStruct((T, 1), jnp.float32),
            jax.ShapeDtypeStruct((T, 1), jnp.float32),
        ],
    )(x, router_w, _REP, _EIDX, _R256, _RSUM, _L128, _L8, _TILE8, _REPT,
      _GSTART)
    pos_flat = pos_qp.reshape(A)

    sh = pl.pallas_call(
        _shared_body,
        grid=(T // MF,),
        in_specs=[
            pl.BlockSpec((MF, D), lambda i: (i, 0)),
            pl.BlockSpec((D, F), lambda i: (0, 0)),
            pl.BlockSpec((D, F), lambda i: (0, 0)),
            pl.BlockSpec((F, D), lambda i: (0, 0)),
        ],
        out_specs=pl.BlockSpec((MF, D), lambda i: (i, 0)),
        out_shape=jax.ShapeDtypeStruct((T, D), jnp.float32),
        compiler_params=pltpu.CompilerParams(
            dimension_semantics=("parallel",),
        ),
    )(x, Ws_gate, Ws_up, Ws_down)

    _dispatch, _collect = _sc_kernels()
    xs = _dispatch(pos_flat, x)

    y = pl.pallas_call(
        _grouped_body,
        grid_spec=pltpu.PrefetchScalarGridSpec(
            num_scalar_prefetch=2,
            grid=(NT,),
            in_specs=[
                pl.BlockSpec((MG, D), lambda i, ex, vl: (i, 0)),
                pl.BlockSpec((1, D, F), lambda i, ex, vl: (ex[i, 0], 0, 0)),
                pl.BlockSpec((1, D, F), lambda i, ex, vl: (ex[i, 0], 0, 0)),
                pl.BlockSpec((1, F, D), lambda i, ex, vl: (ex[i, 0], 0, 0)),
            ],
            out_specs=pl.BlockSpec((MG, D), lambda i, ex, vl: (i, 0)),
        ),
        out_shape=jax.ShapeDtypeStruct((RBUF, D), jnp.float32),
        compiler_params=pltpu.CompilerParams(
            dimension_semantics=("arbitrary",),
        ),
    )(ex_tile, val_tile, xs, W_gate, W_up, W_down)

    yall = _collect(pos_flat, y)

    nf = T // MF
    out = pl.pallas_call(
        _final_body,
        grid=(nf,),
        in_specs=[
            pl.BlockSpec((MF, D), lambda i: (i, 0)),
            pl.BlockSpec((MF, D), lambda i: (i, 0)),
            pl.BlockSpec((MF, D), lambda i: (i + nf, 0)),
            pl.BlockSpec((MF, 1), lambda i: (i, 0)),
            pl.BlockSpec((MF, 1), lambda i: (i, 0)),
        ],
        out_specs=pl.BlockSpec((MF, D), lambda i: (i, 0)),
        out_shape=jax.ShapeDtypeStruct((T, D), jnp.float32),
        compiler_params=pltpu.CompilerParams(
            dimension_semantics=("parallel",),
        ),
    )(sh, yall, yall, w1, w2)
    return out


# speedup vs baseline: 1.0100x; 1.0052x over previous
import functools

import numpy as np
import jax
import jax.numpy as jnp
from jax import lax
from jax.experimental import pallas as pl
from jax.experimental.pallas import tpu as pltpu
from jax.experimental.pallas import tpu_sc as plsc

T = 2048
D = 768
F = 1280
E = 8
K = 2
A = T * K
MG = 512
RBUF = A + E * MG
NT = RBUF // MG
NW = 32
CHUNK = A // NW
MF = 512

_P, _Q = 32, 128
_C = _P * E
_cc = np.arange(_C)
_REP = (np.arange(_P)[:, None] == (_cc // E)[None, :]).astype(np.float32)
_EIDX = (_cc % E).astype(np.float32)[None, :]
_R256 = (((_cc % E)[:, None] == (_cc % E)[None, :])
         & ((_cc // E)[:, None] < (_cc // E)[None, :])).astype(np.float32)
_RSUM = ((_cc % E)[:, None] == np.arange(E)[None, :]).astype(np.float32)
_L128 = (np.arange(_Q)[None, :] < np.arange(_Q)[:, None]).astype(np.float32)
_L8 = (np.arange(E)[:, None] < np.arange(E)[None, :]).astype(np.float32)
_TILE8 = (np.arange(E)[:, None] == (_cc % E)[None, :]).astype(np.float32)
_REPT = ((_cc // E)[:, None] == np.arange(_P)[None, :]).astype(np.float32)

_GSTART = (np.arange(NT) * MG).astype(np.float32)[:, None]


def _route_index_body(x_ref, rw_ref, rep_ref, eidx_ref, r_ref, rsum_ref,
                      l128_ref, l8_ref, tile8_ref, rept_ref, gs_ref,
                      pos_ref, ex_ref, val_ref, w1_ref, w2_ref):
    x = x_ref[...]
    logits = jnp.dot(x, rw_ref[...], preferred_element_type=jnp.float32)
    aff = jax.nn.sigmoid(logits)
    ii = jax.lax.broadcasted_iota(jnp.int32, aff.shape, 1)
    m1 = jnp.max(aff, axis=1, keepdims=True)
    e1 = jnp.min(jnp.where(aff == m1, ii, E), axis=1, keepdims=True)
    aff2 = jnp.where(ii == e1, -1.0, aff)
    m2 = jnp.max(aff2, axis=1, keepdims=True)
    e2 = jnp.min(jnp.where(aff2 == m2, ii, E), axis=1, keepdims=True)
    denom = m1 + m2
    w1_ref[...] = m1 / denom
    w2_ref[...] = m2 / denom

    hi = jax.lax.Precision.HIGHEST
    dot = functools.partial(jnp.dot, precision=hi,
                            preferred_element_type=jnp.float32)
    ef = jnp.concatenate(
        [jnp.reshape(e1, (_Q // 2, _P)), jnp.reshape(e2, (_Q // 2, _P))],
        axis=0).astype(jnp.float32)
    eexp = dot(ef, rep_ref[...])
    z = (eexp == eidx_ref[...]).astype(jnp.float32)
    intra = dot(z, r_ref[...])
    rowcnt = dot(z, rsum_ref[...])
    rowoff = dot(l128_ref[...], rowcnt)
    counts = jnp.sum(rowcnt, axis=0, keepdims=True)
    pc = jnp.floor((counts + (MG - 1)) * (1.0 / MG)) * MG
    pad_off = dot(pc, l8_ref[...])
    off_exp = dot(rowoff + pad_off, tile8_ref[...])
    pos_sel = dot(z * (intra + off_exp), rept_ref[...])
    pos_ref[...] = pos_sel.astype(jnp.int32)

    pad_end = pad_off + pc
    gs = gs_ref[...]
    ex = jnp.sum((gs >= pad_end).astype(jnp.float32), axis=1, keepdims=True)
    ex_ref[...] = jnp.clip(ex, 0.0, float(E - 1)).astype(jnp.int32)
    val_ref[...] = (gs < pad_end[:, E - 1:E]).astype(jnp.int32)


@functools.lru_cache(maxsize=1)
def _sc_kernels():
    mesh = plsc.VectorSubcoreMesh(core_axis_name="c", subcore_axis_name="s")

    H = CHUNK // 2

    @functools.partial(
        pl.kernel,
        out_type=jax.ShapeDtypeStruct((RBUF, D), jnp.float32),
        mesh=mesh,
        scratch_types=[
            pltpu.VMEM((2, H), jnp.int32),
            pltpu.VMEM((2, H, D), jnp.float32),
            pltpu.SemaphoreType.DMA,
            pltpu.SemaphoreType.DMA,
            pltpu.SemaphoreType.DMA,
        ],
    )
    def _dispatch(pos_hbm, x_hbm, xs_hbm, pos_v, rows_v, psem, gsem, ssem):
        wid = lax.axis_index("s") * 2 + lax.axis_index("c")
        base = wid * CHUNK
        tbase = lax.rem(base, T)
        cp0 = pltpu.async_copy(pos_hbm.at[pl.ds(base, H)], pos_v.at[0], psem)
        cp1 = pltpu.async_copy(pos_hbm.at[pl.ds(base + H, H)], pos_v.at[1],
                               psem)
        g0 = pltpu.async_copy(x_hbm.at[pl.ds(tbase, H)], rows_v.at[0], gsem)
        g1 = pltpu.async_copy(x_hbm.at[pl.ds(tbase + H, H)], rows_v.at[1],
                              gsem)
        cp0.wait()
        g0.wait()
        s0 = pltpu.async_copy(rows_v.at[0], xs_hbm.at[pos_v.at[0]], ssem)
        cp1.wait()
        g1.wait()
        s1 = pltpu.async_copy(rows_v.at[1], xs_hbm.at[pos_v.at[1]], ssem)
        s0.wait()
        s1.wait()

    @functools.partial(
        pl.kernel,
        out_type=jax.ShapeDtypeStruct((A, D), jnp.float32),
        mesh=mesh,
        scratch_types=[
            pltpu.VMEM((2, H), jnp.int32),
            pltpu.VMEM((2, H, D), jnp.float32),
            pltpu.SemaphoreType.DMA,
            pltpu.SemaphoreType.DMA,
            pltpu.SemaphoreType.DMA,
        ],
    )
    def _collect(pos_hbm, y_hbm, yall_hbm, pos_v, rows_v, psem, gsem, ssem):
        wid = lax.axis_index("s") * 2 + lax.axis_index("c")
        base = wid * CHUNK
        cp0 = pltpu.async_copy(pos_hbm.at[pl.ds(base, H)], pos_v.at[0], psem)
        cp1 = pltpu.async_copy(pos_hbm.at[pl.ds(base + H, H)], pos_v.at[1],
                               psem)
        cp0.wait()
        g0 = pltpu.async_copy(y_hbm.at[pos_v.at[0]], rows_v.at[0], gsem)
        cp1.wait()
        g1 = pltpu.async_copy(y_hbm.at[pos_v.at[1]], rows_v.at[1], gsem)
        g0.wait()
        s0 = pltpu.async_copy(rows_v.at[0],
                              yall_hbm.at[pl.ds(base, H)], ssem)
        g1.wait()
        s1 = pltpu.async_copy(rows_v.at[1],
                              yall_hbm.at[pl.ds(base + H, H)], ssem)
        s0.wait()
        s1.wait()

    return _dispatch, _collect


def _grouped_body(ex_ref, val_ref, xs_ref, wg_ref, wu_ref, wd_ref, y_ref):
    s = pl.program_id(0)

    @pl.when(val_ref[s, 0] == 1)
    def _expert():
        x = xs_ref[...]
        g = jax.nn.silu(jnp.dot(x, wg_ref[0],
                                preferred_element_type=jnp.float32))
        u = jnp.dot(x, wu_ref[0], preferred_element_type=jnp.float32)
        y_ref[...] = jnp.dot(g * u, wd_ref[0],
                             preferred_element_type=jnp.float32)


def _shared_body(x_ref, wsg_ref, wsu_ref, wsd_ref, out_ref):
    x = x_ref[...]
    g = jax.nn.silu(jnp.dot(x, wsg_ref[...],
                            preferred_element_type=jnp.float32))
    u = jnp.dot(x, wsu_ref[...], preferred_element_type=jnp.float32)
    out_ref[...] = jnp.dot(g * u, wsd_ref[...],
                           preferred_element_type=jnp.float32)


def _final_body(sh_ref, y0_ref, y1_ref, w1_ref, w2_ref, out_ref):
    out_ref[...] = (sh_ref[...] + w1_ref[...] * y0_ref[...]
                    + w2_ref[...] * y1_ref[...])


@jax.jit
def kernel(x, router_w, W_gate, W_up, W_down, Ws_gate, Ws_up, Ws_down):
    full = lambda s: pl.BlockSpec(s, lambda i: tuple(0 for _ in s))
    pos_qp, ex_tile, val_tile, w1, w2 = pl.pallas_call(
        _route_index_body,
        grid=(1,),
        in_specs=[full((T, D)), full((D, E)), full((_P, _C)), full((1, _C)),
                  full((_C, _C)), full((_C, E)), full((_Q, _Q)),
                  full((E, E)), full((E, _C)), full((_C, _P)),
                  full((NT, 1))],
        out_specs=[full((_Q, _P)), full((NT, 1)), full((NT, 1)),
                   full((T, 1)), full((T, 1))],
        out_shape=[
            jax.ShapeDtypeStruct((_Q, _P), jnp.int32),
            jax.ShapeDtypeStruct((NT, 1), jnp.int32),
            jax.ShapeDtypeStruct((NT, 1), jnp.int32),
            jax.ShapeDtypeStruct((T, 1), jnp.float32),
            jax.ShapeDtypeStruct((T, 1), jnp.float32),
        ],
    )(x, router_w, _REP, _EIDX, _R256, _RSUM, _L128, _L8, _TILE8, _REPT,
      _GSTART)
    pos_flat = pos_qp.reshape(A)

    sh = pl.pallas_call(
        _shared_body,
        grid=(T // MF,),
        in_specs=[
            pl.BlockSpec((MF, D), lambda i: (i, 0)),
            pl.BlockSpec((D, F), lambda i: (0, 0)),
            pl.BlockSpec((D, F), lambda i: (0, 0)),
            pl.BlockSpec((F, D), lambda i: (0, 0)),
        ],
        out_specs=pl.BlockSpec((MF, D), lambda i: (i, 0)),
        out_shape=jax.ShapeDtypeStruct((T, D), jnp.float32),
        compiler_params=pltpu.CompilerParams(
            dimension_semantics=("parallel",),
        ),
    )(x, Ws_gate, Ws_up, Ws_down)

    _dispatch, _collect = _sc_kernels()
    xs = _dispatch(pos_flat, x)

    y = pl.pallas_call(
        _grouped_body,
        grid_spec=pltpu.PrefetchScalarGridSpec(
            num_scalar_prefetch=2,
            grid=(NT,),
            in_specs=[
                pl.BlockSpec((MG, D), lambda i, ex, vl: (i, 0)),
                pl.BlockSpec((1, D, F), lambda i, ex, vl: (ex[i, 0], 0, 0)),
                pl.BlockSpec((1, D, F), lambda i, ex, vl: (ex[i, 0], 0, 0)),
                pl.BlockSpec((1, F, D), lambda i, ex, vl: (ex[i, 0], 0, 0)),
            ],
            out_specs=pl.BlockSpec((MG, D), lambda i, ex, vl: (i, 0)),
        ),
        out_shape=jax.ShapeDtypeStruct((RBUF, D), jnp.float32),
        compiler_params=pltpu.CompilerParams(
            dimension_semantics=("arbitrary",),
        ),
    )(ex_tile, val_tile, xs, W_gate, W_up, W_down)

    yall = _collect(pos_flat, y)

    nf = T // MF
    out = pl.pallas_call(
        _final_body,
        grid=(nf,),
        in_specs=[
            pl.BlockSpec((MF, D), lambda i: (i, 0)),
            pl.BlockSpec((MF, D), lambda i: (i, 0)),
            pl.BlockSpec((MF, D), lambda i: (i + nf, 0)),
            pl.BlockSpec((MF, 1), lambda i: (i, 0)),
            pl.BlockSpec((MF, 1), lambda i: (i, 0)),
        ],
        out_specs=pl.BlockSpec((MF, D), lambda i: (i, 0)),
        out_shape=jax.ShapeDtypeStruct((T, D), jnp.float32),
        compiler_params=pltpu.CompilerParams(
            dimension_semantics=("parallel",),
        ),
    )(sh, yall, yall, w1, w2)
    return out
